# baseline (speedup 1.0000x reference)
import jax
import jax.numpy as jnp
from jax import lax
from jax.experimental import pallas as pl
from jax.experimental.pallas import tpu as pltpu
from jax.experimental.pallas import tpu_sc as plsc

N = 10000
E = 320000
D = 128
G = 64
OUT = 10

NH = N // 2
AROWS = NH + 8
NS = 16
EPS = E // NS
C = 80
K = 250
KP = 256
RPS = 320
ZC = 8

CH = 1000
GRID = N // CH

_HI = jax.lax.Precision.HIGHEST


def _sc_agg_body(h_hbm, src_hbm, dst0_hbm, dst1_hbm, out_hbm,
                 src_v, dst_v, rows0_v, rows1_v, wbuf_v,
                 offv, acc, sg0, sg1):
  c = lax.axis_index("c")
  s = lax.axis_index("s")
  nz = jnp.where(s == 15, 25, 40)

  @pl.loop(0, ZC)
  def _(i):
    @pl.loop(0, D, step=16)
    def _(j):
      wbuf_v[i, pl.ds(j, 16)] = jnp.zeros((16,), jnp.float32)

  @pl.loop(0, nz)
  def _(r):
    pltpu.sync_copy(wbuf_v, acc.at[pl.ds(s * RPS + r * ZC, ZC)])

  pltpu.sync_copy(src_hbm.at[pl.ds(s * KP, KP)], src_v)

  @pl.when(c == 0)
  def _():
    pltpu.sync_copy(dst0_hbm.at[pl.ds(s * KP, KP)], dst_v)

  @pl.when(c == 1)
  def _():
    pltpu.sync_copy(dst1_hbm.at[pl.ds(s * KP, KP)], dst_v)

  zero16 = jnp.zeros((16,), jnp.int32)
  one16 = zero16 + 1
  offv[...] = zero16

  @pl.loop(0, K)
  def _(r):
    off = offv[...]
    for j in range(0, C, 16):
      s16 = src_v[r, pl.ds(j, 16)]
      ld16 = dst_v[r, pl.ds(j, 16)]
      m = ld16 < NH
      pos = off + plsc.cumsum(one16, mask=m) - 1
      pr = pos // C
      pc = pos - pr * C
      plsc.store_scatter(src_v, [pr, pc], s16, mask=m)
      plsc.store_scatter(dst_v, [pr, pc], ld16, mask=m)
      off = off + plsc.all_reduce_population_count(m)
    offv[...] = off

  off = offv[...]

  iota16 = lax.iota(jnp.int32, 16)
  trash16 = zero16 + NH
  for p in range(C // 16):
    pos = off + iota16 + (16 * p)
    pr = pos // C
    pc = pos - pr * C
    plsc.store_scatter(src_v, [pr, pc], zero16)
    plsc.store_scatter(dst_v, [pr, pc], trash16)
  cnt = jnp.max(off)
  nk = (cnt + (C - 1)) // C

  plsc.subcore_barrier()

  @pl.when(nk >= 1)
  def _():
    pltpu.async_copy(h_hbm.at[src_v.at[0]], rows0_v, sg0)

  @pl.when(nk >= 2)
  def _():
    pltpu.async_copy(h_hbm.at[src_v.at[1]], rows1_v, sg1)

  @pl.loop(0, nk, step=2)
  def _(k):
    pltpu.make_async_copy(h_hbm.at[src_v.at[k]], rows0_v, sg0).wait()
    pltpu.sync_copy(rows0_v, acc.at[dst_v.at[k]], add=True)

    @pl.when(k + 2 < nk)
    def _():
      pltpu.async_copy(h_hbm.at[src_v.at[k + 2]], rows0_v, sg0)

    @pl.when(k + 1 < nk)
    def _():
      pltpu.make_async_copy(h_hbm.at[src_v.at[k + 1]], rows1_v, sg1).wait()
      pltpu.sync_copy(rows1_v, acc.at[dst_v.at[k + 1]], add=True)

      @pl.when(k + 3 < nk)
      def _():
        pltpu.async_copy(h_hbm.at[src_v.at[k + 3]], rows1_v, sg1)

  plsc.subcore_barrier()

  @pl.loop(0, nz)
  def _(r):
    base = s * RPS + r * ZC
    pltpu.sync_copy(acc.at[pl.ds(base, ZC)], wbuf_v)
    pltpu.sync_copy(wbuf_v, out_hbm.at[pl.ds(c * NH + base, ZC)])


def _sc_aggregate(h, src2d, dst0_2d, dst1_2d):
  mesh = plsc.VectorSubcoreMesh(core_axis_name="c", subcore_axis_name="s")
  kern = pl.kernel(
      _sc_agg_body,
      out_type=jax.ShapeDtypeStruct((N, D), jnp.float32),
      mesh=mesh,
      compiler_params=pltpu.CompilerParams(needs_layout_passes=False),
      scratch_types=[
          pltpu.VMEM((KP, C), jnp.int32),
          pltpu.VMEM((KP, C), jnp.int32),
          pltpu.VMEM((C, D), jnp.float32),
          pltpu.VMEM((C, D), jnp.float32),
          pltpu.VMEM((ZC, D), jnp.float32),
          pltpu.VMEM((16,), jnp.int32),
          pltpu.VMEM_SHARED((AROWS, D), jnp.float32),
          pltpu.SemaphoreType.DMA,
          pltpu.SemaphoreType.DMA,
      ],
  )
  return kern(h, src2d, dst0_2d, dst1_2d)


def _combine_body(p, x, wr, wo, b, o):
  t = jnp.dot(p[...], wr[...], precision=_HI,
              preferred_element_type=jnp.float32)
  t += jnp.dot(x[...], wo[...], precision=_HI,
               preferred_element_type=jnp.float32)
  o[...] = jnp.tanh(t + b[...])


def _tc_combine(p, x, wr, wo, b):
  row = pl.BlockSpec((CH, D), lambda i: (i, 0))
  full = pl.BlockSpec((D, D), lambda i: (0, 0))
  bias = pl.BlockSpec((1, D), lambda i: (0, 0))
  return pl.pallas_call(
      _combine_body,
      grid=(GRID,),
      in_specs=[row, row, full, full, bias],
      out_specs=row,
      out_shape=jax.ShapeDtypeStruct((N, D), jnp.float32),
  )(p, x, wr, wo, b.reshape(1, D))


def _final_body(p, x, wr, wo, b, bt, wm1, bm1, wm2, bm2, o, pooled):
  i = pl.program_id(0)
  t = jnp.dot(p[...], wr[...], precision=_HI,
              preferred_element_type=jnp.float32)
  t += jnp.dot(x[...], wo[...], precision=_HI,
               preferred_element_type=jnp.float32)
  h2 = jnp.tanh(t + b[...])
  gids = lax.broadcasted_iota(jnp.int32, (G, CH), 0)
  onehot = (gids == bt[0]).astype(jnp.float32)
  part = jnp.dot(onehot, h2, precision=_HI, preferred_element_type=jnp.float32)

  @pl.when(i == 0)
  def _():
    pooled[...] = part

  @pl.when(i > 0)
  def _():
    pooled[...] += part

  @pl.when(i == GRID - 1)
  def _():
    hid = jnp.dot(pooled[...], wm1[...], precision=_HI,
                  preferred_element_type=jnp.float32) + bm1[...]
    hid = jnp.maximum(hid, 0.0)
    logit = jnp.dot(hid, wm2[...], precision=_HI,
                    preferred_element_type=jnp.float32) + bm2[...]
    o[...] = jax.nn.sigmoid(logit)


def _tc_final(p, h1, wr, wo, b, batch3d, wm1, bm1, wm2, bm2):
  row = pl.BlockSpec((CH, D), lambda i: (i, 0))
  full = pl.BlockSpec((D, D), lambda i: (0, 0))
  bias = pl.BlockSpec((1, D), lambda i: (0, 0))
  bspec = pl.BlockSpec((1, 1, CH), lambda i: (i, 0, 0))
  wm2s = pl.BlockSpec((D, OUT), lambda i: (0, 0))
  bm2s = pl.BlockSpec((1, OUT), lambda i: (0, 0))
  ospec = pl.BlockSpec((G, OUT), lambda i: (0, 0))
  return pl.pallas_call(
      _final_body,
      grid=(GRID,),
      in_specs=[row, row, full, full, bias, bspec, full, bias, wm2s, bm2s],
      out_specs=ospec,
      out_shape=jax.ShapeDtypeStruct((G, OUT), jnp.float32),
      scratch_shapes=[pltpu.VMEM((G, D), jnp.float32)],
  )(p, h1, wr, wo, b.reshape(1, D), batch3d, wm1, bm1.reshape(1, D),
    wm2, bm2.reshape(1, OUT))


def kernel(x, edge_index, batch, W_rel0, W_root0, b0, W_rel1, W_root1, b1,
           Wm1, bm1, Wm2, bm2):
  src = edge_index[0]
  dst = edge_index[1]
  dst0 = jnp.where(dst < NH, dst, NH)
  dst1 = jnp.where(dst >= NH, dst - NH, NH)

  def chunked(v, pad_val):
    v2 = v.reshape(NS, EPS)
    v2 = jnp.pad(v2, ((0, 0), (0, KP * C - EPS)),
                 constant_values=pad_val)
    return v2.reshape(NS * KP, C)

  src2d = chunked(src, 0)
  dst0_2d = chunked(dst0, NH)
  dst1_2d = chunked(dst1, NH)
  batch3d = batch.reshape(GRID, 1, CH)

  a0 = _sc_aggregate(x, src2d, dst0_2d, dst1_2d)
  h1 = _tc_combine(a0, x, W_rel0, W_root0, b0)
  a1 = _sc_aggregate(h1, src2d, dst0_2d, dst1_2d)
  out = _tc_final(a1, h1, W_rel1, W_root1, b1, batch3d,
                  Wm1, bm1, Wm2, bm2)
  return out


# speedup vs baseline: 1.1379x; 1.1379x over previous
import jax
import jax.numpy as jnp
from jax import lax
from jax.experimental import pallas as pl
from jax.experimental.pallas import tpu as pltpu
from jax.experimental.pallas import tpu_sc as plsc

N = 10000
E = 320000
D = 128
G = 64
OUT = 10

NH = N // 2
AROWS = NH + 8
NS = 16
EPS = E // NS
C = 80
K = 250
KP = 256
RPS = 320
ZC = 8

CH = 1000
GRID = N // CH

_HI = jax.lax.Precision.HIGHEST


def _zero_acc(wbuf_v, acc, s, nz):
  @pl.loop(0, ZC)
  def _(i):
    @pl.loop(0, D, step=16)
    def _(j):
      wbuf_v[i, pl.ds(j, 16)] = jnp.zeros((16,), jnp.float32)

  @pl.loop(0, nz)
  def _(r):
    pltpu.sync_copy(wbuf_v, acc.at[pl.ds(s * RPS + r * ZC, ZC)])


def _gather_scatter(h_hbm, src_v, dst_v, rows0_v, rows1_v, acc,
                    sg0, sg1, nk):
  @pl.when(nk >= 1)
  def _():
    pltpu.async_copy(h_hbm.at[src_v.at[0]], rows0_v, sg0)

  @pl.when(nk >= 2)
  def _():
    pltpu.async_copy(h_hbm.at[src_v.at[1]], rows1_v, sg1)

  @pl.loop(0, nk, step=2)
  def _(k):
    pltpu.make_async_copy(h_hbm.at[src_v.at[k]], rows0_v, sg0).wait()
    pltpu.sync_copy(rows0_v, acc.at[dst_v.at[k]], add=True)

    @pl.when(k + 2 < nk)
    def _():
      pltpu.async_copy(h_hbm.at[src_v.at[k + 2]], rows0_v, sg0)

    @pl.when(k + 1 < nk)
    def _():
      pltpu.make_async_copy(h_hbm.at[src_v.at[k + 1]], rows1_v, sg1).wait()
      pltpu.sync_copy(rows1_v, acc.at[dst_v.at[k + 1]], add=True)

      @pl.when(k + 3 < nk)
      def _():
        pltpu.async_copy(h_hbm.at[src_v.at[k + 3]], rows1_v, sg1)


def _writeout(out_hbm, acc, wbuf_v, c, s, nz):
  @pl.loop(0, nz)
  def _(r):
    base = s * RPS + r * ZC
    pltpu.sync_copy(acc.at[pl.ds(base, ZC)], wbuf_v)
    pltpu.sync_copy(wbuf_v, out_hbm.at[pl.ds(c * NH + base, ZC)])


def _sc_compact_agg_body(h_hbm, src_hbm, dst0_hbm, dst1_hbm,
                         out_hbm, csrc_hbm, cdst_hbm, cnt_hbm,
                         src_v, dst_v, rows0_v, rows1_v, wbuf_v,
                         offv, acc, sg0, sg1):
  c = lax.axis_index("c")
  s = lax.axis_index("s")
  nz = jnp.where(s == 15, 25, 40)

  _zero_acc(wbuf_v, acc, s, nz)

  pltpu.sync_copy(src_hbm.at[pl.ds(s * KP, KP)], src_v)

  @pl.when(c == 0)
  def _():
    pltpu.sync_copy(dst0_hbm.at[pl.ds(s * KP, KP)], dst_v)

  @pl.when(c == 1)
  def _():
    pltpu.sync_copy(dst1_hbm.at[pl.ds(s * KP, KP)], dst_v)

  zero16 = jnp.zeros((16,), jnp.int32)
  one16 = zero16 + 1
  offv[...] = zero16

  @pl.loop(0, K)
  def _(r):
    off = offv[...]
    for j in range(0, C, 16):
      s16 = src_v[r, pl.ds(j, 16)]
      ld16 = dst_v[r, pl.ds(j, 16)]
      m = ld16 < NH
      pos = off + plsc.cumsum(one16, mask=m) - 1
      pr = pos // C
      pc = pos - pr * C
      plsc.store_scatter(src_v, [pr, pc], s16, mask=m)
      plsc.store_scatter(dst_v, [pr, pc], ld16, mask=m)
      off = off + plsc.all_reduce_population_count(m)
    offv[...] = off

  off = offv[...]

  iota16 = lax.iota(jnp.int32, 16)
  trash16 = zero16 + NH
  for p in range(C // 16):
    pos = off + iota16 + (16 * p)
    pr = pos // C
    pc = pos - pr * C
    plsc.store_scatter(src_v, [pr, pc], zero16)
    plsc.store_scatter(dst_v, [pr, pc], trash16)
  cnt = jnp.max(off)
  nk = (cnt + (C - 1)) // C

  pltpu.sync_copy(src_v, csrc_hbm.at[c, pl.ds(s * KP, KP)])
  pltpu.sync_copy(dst_v, cdst_hbm.at[c, pl.ds(s * KP, KP)])
  pltpu.sync_copy(offv, cnt_hbm.at[c, pl.ds(s * 16, 16)])

  plsc.subcore_barrier()
  _gather_scatter(h_hbm, src_v, dst_v, rows0_v, rows1_v, acc, sg0, sg1, nk)
  plsc.subcore_barrier()
  _writeout(out_hbm, acc, wbuf_v, c, s, nz)


def _sc_pre_agg_body(h_hbm, csrc_hbm, cdst_hbm, cnt_hbm, out_hbm,
                     src_v, dst_v, rows0_v, rows1_v, wbuf_v,
                     offv, acc, sg0, sg1):
  c = lax.axis_index("c")
  s = lax.axis_index("s")
  nz = jnp.where(s == 15, 25, 40)

  _zero_acc(wbuf_v, acc, s, nz)

  pltpu.sync_copy(csrc_hbm.at[c, pl.ds(s * KP, KP)], src_v)
  pltpu.sync_copy(cdst_hbm.at[c, pl.ds(s * KP, KP)], dst_v)
  pltpu.sync_copy(cnt_hbm.at[c, pl.ds(s * 16, 16)], offv)
  cnt = jnp.max(offv[...])
  nk = (cnt + (C - 1)) // C

  plsc.subcore_barrier()
  _gather_scatter(h_hbm, src_v, dst_v, rows0_v, rows1_v, acc, sg0, sg1, nk)
  plsc.subcore_barrier()
  _writeout(out_hbm, acc, wbuf_v, c, s, nz)


_SC_SCRATCH = [
    pltpu.VMEM((KP, C), jnp.int32),
    pltpu.VMEM((KP, C), jnp.int32),
    pltpu.VMEM((C, D), jnp.float32),
    pltpu.VMEM((C, D), jnp.float32),
    pltpu.VMEM((ZC, D), jnp.float32),
    pltpu.VMEM((16,), jnp.int32),
    pltpu.VMEM_SHARED((AROWS, D), jnp.float32),
    pltpu.SemaphoreType.DMA,
    pltpu.SemaphoreType.DMA,
]


def _sc_compact_aggregate(h, src2d, dst0_2d, dst1_2d):
  mesh = plsc.VectorSubcoreMesh(core_axis_name="c", subcore_axis_name="s")
  kern = pl.kernel(
      _sc_compact_agg_body,
      out_type=[
          jax.ShapeDtypeStruct((N, D), jnp.float32),
          jax.ShapeDtypeStruct((2, NS * KP, C), jnp.int32),
          jax.ShapeDtypeStruct((2, NS * KP, C), jnp.int32),
          jax.ShapeDtypeStruct((2, NS * 16), jnp.int32),
      ],
      mesh=mesh,
      compiler_params=pltpu.CompilerParams(needs_layout_passes=False),
      scratch_types=list(_SC_SCRATCH),
  )
  return kern(h, src2d, dst0_2d, dst1_2d)


def _sc_pre_aggregate(h, csrc, cdst, cnts):
  mesh = plsc.VectorSubcoreMesh(core_axis_name="c", subcore_axis_name="s")
  kern = pl.kernel(
      _sc_pre_agg_body,
      out_type=jax.ShapeDtypeStruct((N, D), jnp.float32),
      mesh=mesh,
      compiler_params=pltpu.CompilerParams(needs_layout_passes=False),
      scratch_types=list(_SC_SCRATCH),
  )
  return kern(h, csrc, cdst, cnts)


def _combine_body(p, x, wr, wo, b, o):
  t = jnp.dot(p[...], wr[...], precision=_HI,
              preferred_element_type=jnp.float32)
  t += jnp.dot(x[...], wo[...], precision=_HI,
               preferred_element_type=jnp.float32)
  o[...] = jnp.tanh(t + b[...])


def _tc_combine(p, x, wr, wo, b):
  row = pl.BlockSpec((CH, D), lambda i: (i, 0))
  full = pl.BlockSpec((D, D), lambda i: (0, 0))
  bias = pl.BlockSpec((1, D), lambda i: (0, 0))
  return pl.pallas_call(
      _combine_body,
      grid=(GRID,),
      in_specs=[row, row, full, full, bias],
      out_specs=row,
      out_shape=jax.ShapeDtypeStruct((N, D), jnp.float32),
  )(p, x, wr, wo, b.reshape(1, D))


def _final_body(p, x, wr, wo, b, bt, wm1, bm1, wm2, bm2, o, pooled):
  i = pl.program_id(0)
  t = jnp.dot(p[...], wr[...], precision=_HI,
              preferred_element_type=jnp.float32)
  t += jnp.dot(x[...], wo[...], precision=_HI,
               preferred_element_type=jnp.float32)
  h2 = jnp.tanh(t + b[...])
  gids = lax.broadcasted_iota(jnp.int32, (G, CH), 0)
  onehot = (gids == bt[0]).astype(jnp.float32)
  part = jnp.dot(onehot, h2, precision=_HI, preferred_element_type=jnp.float32)

  @pl.when(i == 0)
  def _():
    pooled[...] = part

  @pl.when(i > 0)
  def _():
    pooled[...] += part

  @pl.when(i == GRID - 1)
  def _():
    hid = jnp.dot(pooled[...], wm1[...], precision=_HI,
                  preferred_element_type=jnp.float32) + bm1[...]
    hid = jnp.maximum(hid, 0.0)
    logit = jnp.dot(hid, wm2[...], precision=_HI,
                    preferred_element_type=jnp.float32) + bm2[...]
    o[...] = jax.nn.sigmoid(logit)


def _tc_final(p, h1, wr, wo, b, batch3d, wm1, bm1, wm2, bm2):
  row = pl.BlockSpec((CH, D), lambda i: (i, 0))
  full = pl.BlockSpec((D, D), lambda i: (0, 0))
  bias = pl.BlockSpec((1, D), lambda i: (0, 0))
  bspec = pl.BlockSpec((1, 1, CH), lambda i: (i, 0, 0))
  wm2s = pl.BlockSpec((D, OUT), lambda i: (0, 0))
  bm2s = pl.BlockSpec((1, OUT), lambda i: (0, 0))
  ospec = pl.BlockSpec((G, OUT), lambda i: (0, 0))
  return pl.pallas_call(
      _final_body,
      grid=(GRID,),
      in_specs=[row, row, full, full, bias, bspec, full, bias, wm2s, bm2s],
      out_specs=ospec,
      out_shape=jax.ShapeDtypeStruct((G, OUT), jnp.float32),
      scratch_shapes=[pltpu.VMEM((G, D), jnp.float32)],
  )(p, h1, wr, wo, b.reshape(1, D), batch3d, wm1, bm1.reshape(1, D),
    wm2, bm2.reshape(1, OUT))


def kernel(x, edge_index, batch, W_rel0, W_root0, b0, W_rel1, W_root1, b1,
           Wm1, bm1, Wm2, bm2):
  src = edge_index[0]
  dst = edge_index[1]
  dst0 = jnp.where(dst < NH, dst, NH)
  dst1 = jnp.where(dst >= NH, dst - NH, NH)

  def chunked(v, pad_val):
    v2 = v.reshape(NS, EPS)
    v2 = jnp.pad(v2, ((0, 0), (0, KP * C - EPS)),
                 constant_values=pad_val)
    return v2.reshape(NS * KP, C)

  src2d = chunked(src, 0)
  dst0_2d = chunked(dst0, NH)
  dst1_2d = chunked(dst1, NH)
  batch3d = batch.reshape(GRID, 1, CH)

  a0, csrc, cdst, cnts = _sc_compact_aggregate(x, src2d, dst0_2d, dst1_2d)
  h1 = _tc_combine(a0, x, W_rel0, W_root0, b0)
  a1 = _sc_pre_aggregate(h1, csrc, cdst, cnts)
  out = _tc_final(a1, h1, W_rel1, W_root1, b1, batch3d,
                  Wm1, bm1, Wm2, bm2)
  return out


# speedup vs baseline: 1.1508x; 1.0113x over previous
import jax
import jax.numpy as jnp
from jax import lax
from jax.experimental import pallas as pl
from jax.experimental.pallas import tpu as pltpu
from jax.experimental.pallas import tpu_sc as plsc

N = 10000
E = 320000
D = 128
G = 64
OUT = 10

NH = N // 2
AROWS = NH + 8
NS = 16
EPS = E // NS
C = 80
K = 250
KP = 256
RPS = 320
ZC = 8

CH = 1000
GRID = N // CH

_HI = jax.lax.Precision.HIGHEST


def _zero_acc_start(wbuf_v, acc, s, nz, sgz):
  @pl.loop(0, ZC)
  def _(i):
    @pl.loop(0, D, step=16)
    def _(j):
      wbuf_v[i, pl.ds(j, 16)] = jnp.zeros((16,), jnp.float32)

  @pl.loop(0, nz)
  def _(r):
    pltpu.async_copy(wbuf_v, acc.at[pl.ds(s * RPS + r * ZC, ZC)], sgz)


def _zero_acc_wait(wbuf_v, acc, s, nz, sgz):
  @pl.loop(0, nz)
  def _(r):
    pltpu.make_async_copy(
        wbuf_v, acc.at[pl.ds(s * RPS + r * ZC, ZC)], sgz).wait()


def _gather_scatter(h_hbm, src_v, dst_v, rows0_v, rows1_v, acc,
                    sg0, sg1, nk):
  @pl.when(nk >= 1)
  def _():
    pltpu.async_copy(h_hbm.at[src_v.at[0]], rows0_v, sg0)

  @pl.when(nk >= 2)
  def _():
    pltpu.async_copy(h_hbm.at[src_v.at[1]], rows1_v, sg1)

  @pl.loop(0, nk, step=2)
  def _(k):
    pltpu.make_async_copy(h_hbm.at[src_v.at[k]], rows0_v, sg0).wait()
    pltpu.sync_copy(rows0_v, acc.at[dst_v.at[k]], add=True)

    @pl.when(k + 2 < nk)
    def _():
      pltpu.async_copy(h_hbm.at[src_v.at[k + 2]], rows0_v, sg0)

    @pl.when(k + 1 < nk)
    def _():
      pltpu.make_async_copy(h_hbm.at[src_v.at[k + 1]], rows1_v, sg1).wait()
      pltpu.sync_copy(rows1_v, acc.at[dst_v.at[k + 1]], add=True)

      @pl.when(k + 3 < nk)
      def _():
        pltpu.async_copy(h_hbm.at[src_v.at[k + 3]], rows1_v, sg1)


def _writeout(out_hbm, acc, wbuf_v, c, s, nz):
  @pl.loop(0, nz)
  def _(r):
    base = s * RPS + r * ZC
    pltpu.sync_copy(acc.at[pl.ds(base, ZC)], wbuf_v)
    pltpu.sync_copy(wbuf_v, out_hbm.at[pl.ds(c * NH + base, ZC)])


def _sc_compact_agg_body(h_hbm, src_hbm, dst0_hbm, dst1_hbm,
                         out_hbm, csrc_hbm, cdst_hbm, cnt_hbm,
                         src_v, dst_v, rows0_v, rows1_v, wbuf_v,
                         offv, acc, sg0, sg1, sgz):
  c = lax.axis_index("c")
  s = lax.axis_index("s")
  nz = jnp.where(s == 15, 25, 40)

  _zero_acc_start(wbuf_v, acc, s, nz, sgz)

  pltpu.sync_copy(src_hbm.at[pl.ds(s * KP, KP)], src_v)

  @pl.when(c == 0)
  def _():
    pltpu.sync_copy(dst0_hbm.at[pl.ds(s * KP, KP)], dst_v)

  @pl.when(c == 1)
  def _():
    pltpu.sync_copy(dst1_hbm.at[pl.ds(s * KP, KP)], dst_v)

  zero16 = jnp.zeros((16,), jnp.int32)
  one16 = zero16 + 1
  offv[...] = zero16

  @pl.loop(0, K)
  def _(r):
    off = offv[...]
    for j in range(0, C, 16):
      s16 = src_v[r, pl.ds(j, 16)]
      ld16 = dst_v[r, pl.ds(j, 16)]
      m = ld16 < NH
      pos = off + plsc.cumsum(one16, mask=m) - 1
      pr = pos // C
      pc = pos - pr * C
      plsc.store_scatter(src_v, [pr, pc], s16, mask=m)
      plsc.store_scatter(dst_v, [pr, pc], ld16, mask=m)
      off = off + plsc.all_reduce_population_count(m)
    offv[...] = off

  off = offv[...]

  iota16 = lax.iota(jnp.int32, 16)
  trash16 = zero16 + NH
  for p in range(C // 16):
    pos = off + iota16 + (16 * p)
    pr = pos // C
    pc = pos - pr * C
    plsc.store_scatter(src_v, [pr, pc], zero16)
    plsc.store_scatter(dst_v, [pr, pc], trash16)
  cnt = jnp.max(off)
  nk = (cnt + (C - 1)) // C

  pltpu.sync_copy(src_v, csrc_hbm.at[c, pl.ds(s * KP, KP)])
  pltpu.sync_copy(dst_v, cdst_hbm.at[c, pl.ds(s * KP, KP)])
  pltpu.sync_copy(offv, cnt_hbm.at[c, pl.ds(s * 16, 16)])

  _zero_acc_wait(wbuf_v, acc, s, nz, sgz)
  plsc.subcore_barrier()
  _gather_scatter(h_hbm, src_v, dst_v, rows0_v, rows1_v, acc, sg0, sg1, nk)
  plsc.subcore_barrier()
  _writeout(out_hbm, acc, wbuf_v, c, s, nz)


def _sc_pre_agg_body(h_hbm, csrc_hbm, cdst_hbm, cnt_hbm, out_hbm,
                     src_v, dst_v, rows0_v, rows1_v, wbuf_v,
                     offv, acc, sg0, sg1, sgz):
  c = lax.axis_index("c")
  s = lax.axis_index("s")
  nz = jnp.where(s == 15, 25, 40)

  _zero_acc_start(wbuf_v, acc, s, nz, sgz)

  pltpu.sync_copy(csrc_hbm.at[c, pl.ds(s * KP, KP)], src_v)
  pltpu.sync_copy(cdst_hbm.at[c, pl.ds(s * KP, KP)], dst_v)
  pltpu.sync_copy(cnt_hbm.at[c, pl.ds(s * 16, 16)], offv)
  cnt = jnp.max(offv[...])
  nk = (cnt + (C - 1)) // C

  _zero_acc_wait(wbuf_v, acc, s, nz, sgz)
  plsc.subcore_barrier()
  _gather_scatter(h_hbm, src_v, dst_v, rows0_v, rows1_v, acc, sg0, sg1, nk)
  plsc.subcore_barrier()
  _writeout(out_hbm, acc, wbuf_v, c, s, nz)


_SC_SCRATCH = [
    pltpu.VMEM((KP, C), jnp.int32),
    pltpu.VMEM((KP, C), jnp.int32),
    pltpu.VMEM((C, D), jnp.float32),
    pltpu.VMEM((C, D), jnp.float32),
    pltpu.VMEM((ZC, D), jnp.float32),
    pltpu.VMEM((16,), jnp.int32),
    pltpu.VMEM_SHARED((AROWS, D), jnp.float32),
    pltpu.SemaphoreType.DMA,
    pltpu.SemaphoreType.DMA,
    pltpu.SemaphoreType.DMA,
]


def _sc_compact_aggregate(h, src2d, dst0_2d, dst1_2d):
  mesh = plsc.VectorSubcoreMesh(core_axis_name="c", subcore_axis_name="s")
  kern = pl.kernel(
      _sc_compact_agg_body,
      out_type=[
          jax.ShapeDtypeStruct((N, D), jnp.float32),
          jax.ShapeDtypeStruct((2, NS * KP, C), jnp.int32),
          jax.ShapeDtypeStruct((2, NS * KP, C), jnp.int32),
          jax.ShapeDtypeStruct((2, NS * 16), jnp.int32),
      ],
      mesh=mesh,
      compiler_params=pltpu.CompilerParams(needs_layout_passes=False),
      scratch_types=list(_SC_SCRATCH),
  )
  return kern(h, src2d, dst0_2d, dst1_2d)


def _sc_pre_aggregate(h, csrc, cdst, cnts):
  mesh = plsc.VectorSubcoreMesh(core_axis_name="c", subcore_axis_name="s")
  kern = pl.kernel(
      _sc_pre_agg_body,
      out_type=jax.ShapeDtypeStruct((N, D), jnp.float32),
      mesh=mesh,
      compiler_params=pltpu.CompilerParams(needs_layout_passes=False),
      scratch_types=list(_SC_SCRATCH),
  )
  return kern(h, csrc, cdst, cnts)


def _combine_body(p, x, wr, wo, b, o):
  t = jnp.dot(p[...], wr[...], precision=_HI,
              preferred_element_type=jnp.float32)
  t += jnp.dot(x[...], wo[...], precision=_HI,
               preferred_element_type=jnp.float32)
  o[...] = jnp.tanh(t + b[...])


def _tc_combine(p, x, wr, wo, b):
  row = pl.BlockSpec((CH, D), lambda i: (i, 0))
  full = pl.BlockSpec((D, D), lambda i: (0, 0))
  bias = pl.BlockSpec((1, D), lambda i: (0, 0))
  return pl.pallas_call(
      _combine_body,
      grid=(GRID,),
      in_specs=[row, row, full, full, bias],
      out_specs=row,
      out_shape=jax.ShapeDtypeStruct((N, D), jnp.float32),
  )(p, x, wr, wo, b.reshape(1, D))


def _final_body(p, x, wr, wo, b, bt, wm1, bm1, wm2, bm2, o, pooled):
  i = pl.program_id(0)
  t = jnp.dot(p[...], wr[...], precision=_HI,
              preferred_element_type=jnp.float32)
  t += jnp.dot(x[...], wo[...], precision=_HI,
               preferred_element_type=jnp.float32)
  h2 = jnp.tanh(t + b[...])
  gids = lax.broadcasted_iota(jnp.int32, (G, CH), 0)
  onehot = (gids == bt[0]).astype(jnp.float32)
  part = jnp.dot(onehot, h2, precision=_HI, preferred_element_type=jnp.float32)

  @pl.when(i == 0)
  def _():
    pooled[...] = part

  @pl.when(i > 0)
  def _():
    pooled[...] += part

  @pl.when(i == GRID - 1)
  def _():
    hid = jnp.dot(pooled[...], wm1[...], precision=_HI,
                  preferred_element_type=jnp.float32) + bm1[...]
    hid = jnp.maximum(hid, 0.0)
    logit = jnp.dot(hid, wm2[...], precision=_HI,
                    preferred_element_type=jnp.float32) + bm2[...]
    o[...] = jax.nn.sigmoid(logit)


def _tc_final(p, h1, wr, wo, b, batch3d, wm1, bm1, wm2, bm2):
  row = pl.BlockSpec((CH, D), lambda i: (i, 0))
  full = pl.BlockSpec((D, D), lambda i: (0, 0))
  bias = pl.BlockSpec((1, D), lambda i: (0, 0))
  bspec = pl.BlockSpec((1, 1, CH), lambda i: (i, 0, 0))
  wm2s = pl.BlockSpec((D, OUT), lambda i: (0, 0))
  bm2s = pl.BlockSpec((1, OUT), lambda i: (0, 0))
  ospec = pl.BlockSpec((G, OUT), lambda i: (0, 0))
  return pl.pallas_call(
      _final_body,
      grid=(GRID,),
      in_specs=[row, row, full, full, bias, bspec, full, bias, wm2s, bm2s],
      out_specs=ospec,
      out_shape=jax.ShapeDtypeStruct((G, OUT), jnp.float32),
      scratch_shapes=[pltpu.VMEM((G, D), jnp.float32)],
  )(p, h1, wr, wo, b.reshape(1, D), batch3d, wm1, bm1.reshape(1, D),
    wm2, bm2.reshape(1, OUT))


def kernel(x, edge_index, batch, W_rel0, W_root0, b0, W_rel1, W_root1, b1,
           Wm1, bm1, Wm2, bm2):
  src = edge_index[0]
  dst = edge_index[1]
  dst0 = jnp.where(dst < NH, dst, NH)
  dst1 = jnp.where(dst >= NH, dst - NH, NH)

  def chunked(v, pad_val):
    v2 = v.reshape(NS, EPS)
    v2 = jnp.pad(v2, ((0, 0), (0, KP * C - EPS)),
                 constant_values=pad_val)
    return v2.reshape(NS * KP, C)

  src2d = chunked(src, 0)
  dst0_2d = chunked(dst0, NH)
  dst1_2d = chunked(dst1, NH)
  batch3d = batch.reshape(GRID, 1, CH)

  a0, csrc, cdst, cnts = _sc_compact_aggregate(x, src2d, dst0_2d, dst1_2d)
  h1 = _tc_combine(a0, x, W_rel0, W_root0, b0)
  a1 = _sc_pre_aggregate(h1, csrc, cdst, cnts)
  out = _tc_final(a1, h1, W_rel1, W_root1, b1, batch3d,
                  Wm1, bm1, Wm2, bm2)
  return out
